# Initial kernel scaffold; baseline (speedup 1.0000x reference)
import jax
import jax.numpy as jnp
from jax.experimental import pallas as pl


def kernel(x, weight):
    raise NotImplementedError("write your pallas kernel here")


# speedup vs baseline: 4.8089x; 4.8089x over previous
import functools

import jax
import jax.numpy as jnp
from jax import lax
from jax.experimental import pallas as pl
from jax.experimental.pallas import tpu as pltpu
from jax.experimental.pallas import tpu_sc as plsc

_NC = 2
_NS = 16
_NW = _NC * _NS
_G = 128
_K = 8


def _gather_body(idx_hbm, table_hbm, out_hbm, idx_v, rows_v, sem):
    wid = lax.axis_index("s") * _NC + lax.axis_index("c")
    rows_per_w = idx_hbm.shape[0] // _NW
    base = wid * rows_per_w

    @pl.loop(0, rows_per_w, step=_K)
    def _(g):
        r0 = base + g
        pltpu.sync_copy(idx_hbm.at[pl.ds(r0, _K)], idx_v)
        for j in range(_K):
            pltpu.async_copy(table_hbm.at[idx_v.at[j]], rows_v.at[j], sem)
        for j in range(_K):
            pltpu.make_async_copy(
                table_hbm.at[idx_v.at[j]], rows_v.at[j], sem).wait()
        pltpu.sync_copy(rows_v, out_hbm.at[pl.ds(r0, _K)])


@functools.partial(jax.jit, static_argnums=())
def _embedding_lookup(idx2, weight):
    n_rows = idx2.shape[0]
    d = weight.shape[1]
    run = pl.kernel(
        _gather_body,
        out_type=jax.ShapeDtypeStruct((n_rows, _G, d), jnp.float32),
        mesh=plsc.VectorSubcoreMesh(
            core_axis_name="c", subcore_axis_name="s",
            num_cores=_NC, num_subcores=_NS),
        scratch_types=[
            pltpu.VMEM((_K, _G), jnp.int32),
            pltpu.VMEM((_K, _G, d), jnp.float32),
            pltpu.SemaphoreType.DMA,
        ],
        compiler_params=pltpu.CompilerParams(use_tc_tiling_on_sc=False),
    )
    return run(idx2, weight)


def kernel(x, weight):
    b, s = x.shape
    d = weight.shape[1]
    idx2 = x.reshape(-1, _G).astype(jnp.int32)
    out = _embedding_lookup(idx2, weight)
    return out.reshape(b, s, d)


# speedup vs baseline: 5.0442x; 1.0489x over previous
import functools

import jax
import jax.numpy as jnp
from jax import lax
from jax.experimental import pallas as pl
from jax.experimental.pallas import tpu as pltpu
from jax.experimental.pallas import tpu_sc as plsc

_NC = 2
_NS = 16
_NW = _NC * _NS
_G = 128
_K = 10


def _gather_body(idx_hbm, table_hbm, out_hbm,
                 idx0, idx1, rows0, rows1,
                 isem0, isem1, gsem0, gsem1, osem0, osem1):
    wid = lax.axis_index("s") * _NC + lax.axis_index("c")
    rows_per_w = idx_hbm.shape[0] // _NW
    n_chunks = rows_per_w // _K
    base = wid * rows_per_w

    idx_b = (idx0, idx1)
    rows_b = (rows0, rows1)
    isem = (isem0, isem1)
    gsem = (gsem0, gsem1)
    osem = (osem0, osem1)

    def chunk_start(g):
        return base + g * _K

    def idx_load(g, b):
        return pltpu.async_copy(idx_hbm.at[pl.ds(chunk_start(g), _K)],
                                idx_b[b], isem[b])

    def fire_gathers(g, b):
        for j in range(_K):
            pltpu.async_copy(table_hbm.at[idx_b[b].at[j]],
                             rows_b[b].at[j], gsem[b])

    def wait_gathers(b):
        pltpu.make_async_copy(out_hbm.at[pl.ds(base, _K)],
                              rows_b[b], gsem[b]).wait()

    def store(g, b):
        return pltpu.async_copy(rows_b[b], out_hbm.at[pl.ds(chunk_start(g), _K)],
                                osem[b])

    def wait_store(b):
        pltpu.make_async_copy(rows_b[b],
                              out_hbm.at[pl.ds(base, _K)], osem[b]).wait()

    def wait_idx(b):
        pltpu.make_async_copy(idx_hbm.at[pl.ds(base, _K)],
                              idx_b[b], isem[b]).wait()

    idx_load(0, 0)
    wait_idx(0)
    fire_gathers(0, 0)
    idx_load(1, 1)
    wait_idx(1)
    fire_gathers(1, 1)
    wait_gathers(0)
    store(0, 0)
    idx_load(2, 0)

    @pl.loop(2, n_chunks, step=2)
    def _(t):
        for b in range(2):
            g = t + b
            wait_idx(b)
            wait_store(b)
            fire_gathers(g, b)
            wait_gathers(1 - b)
            store(g - 1, 1 - b)
            r0 = jnp.minimum(chunk_start(g + 1), base + (rows_per_w - _K))
            pltpu.async_copy(idx_hbm.at[pl.ds(r0, _K)], idx_b[1 - b],
                             isem[1 - b])

    b_last = (n_chunks - 1) % 2
    wait_gathers(b_last)
    store(n_chunks - 1, b_last)
    wait_store(1 - b_last)
    wait_store(b_last)
    wait_idx(1 - b_last)


@jax.jit
def _embedding_lookup(idx2, weight):
    n_rows = idx2.shape[0]
    d = weight.shape[1]
    run = pl.kernel(
        _gather_body,
        out_type=jax.ShapeDtypeStruct((n_rows, _G, d), jnp.float32),
        mesh=plsc.VectorSubcoreMesh(
            core_axis_name="c", subcore_axis_name="s",
            num_cores=_NC, num_subcores=_NS),
        scratch_types=[
            pltpu.VMEM((_K, _G), jnp.int32),
            pltpu.VMEM((_K, _G), jnp.int32),
            pltpu.VMEM((_K, _G, d), jnp.float32),
            pltpu.VMEM((_K, _G, d), jnp.float32),
            pltpu.SemaphoreType.DMA,
            pltpu.SemaphoreType.DMA,
            pltpu.SemaphoreType.DMA,
            pltpu.SemaphoreType.DMA,
            pltpu.SemaphoreType.DMA,
            pltpu.SemaphoreType.DMA,
        ],
        compiler_params=pltpu.CompilerParams(use_tc_tiling_on_sc=False),
    )
    return run(idx2, weight)


def kernel(x, weight):
    b, s = x.shape
    d = weight.shape[1]
    idx2 = x.reshape(-1, _G).astype(jnp.int32)
    out = _embedding_lookup(idx2, weight)
    return out.reshape(b, s, d)


# speedup vs baseline: 5.0483x; 1.0008x over previous
import functools

import jax
import jax.numpy as jnp
from jax import lax
from jax.experimental import pallas as pl
from jax.experimental.pallas import tpu as pltpu
from jax.experimental.pallas import tpu_sc as plsc

_NC = 2
_NS = 16
_NW = _NC * _NS
_G = 128
_K = 10


def _gather_body(idx_hbm, table_hbm, out_hbm,
                 idx0, idx1, rows0, rows1,
                 isem0, isem1, gsem0, gsem1, osem0, osem1):
    wid = lax.axis_index("s") * _NC + lax.axis_index("c")
    n_idx_rows = idx_hbm.shape[0]
    rows_per_w = n_idx_rows // _NW
    n_chunks = rows_per_w // _K
    base = wid * rows_per_w
    cs = _K * _G

    idx_b = (idx0, idx1)
    rows_b = (rows0, rows1)
    isem = (isem0, isem1)
    gsem = (gsem0, gsem1)
    osem = (osem0, osem1)

    def idx_start(g):
        return base + g * _K

    def flat_start(g):
        return (base + g * _K) * _G

    def idx_load(g, b):
        pltpu.async_copy(idx_hbm.at[pl.ds(idx_start(g), _K)],
                         idx_b[b], isem[b])

    def fire_gathers(g, b):
        for j in range(_K):
            pltpu.async_copy(table_hbm.at[idx_b[b].at[j]],
                             rows_b[b].at[pl.ds(j * _G, _G)], gsem[b])

    def wait_gathers(b):
        pltpu.make_async_copy(out_hbm.at[pl.ds(0, cs)],
                              rows_b[b], gsem[b]).wait()

    def store(g, b):
        pltpu.async_copy(rows_b[b], out_hbm.at[pl.ds(flat_start(g), cs)],
                         osem[b])

    def wait_store(b):
        pltpu.make_async_copy(rows_b[b],
                              out_hbm.at[pl.ds(0, cs)], osem[b]).wait()

    def wait_idx(b):
        pltpu.make_async_copy(idx_hbm.at[pl.ds(0, _K)],
                              idx_b[b], isem[b]).wait()

    idx_load(0, 0)
    wait_idx(0)
    fire_gathers(0, 0)
    idx_load(1, 1)
    wait_idx(1)
    fire_gathers(1, 1)
    wait_gathers(0)
    store(0, 0)
    idx_load(2, 0)

    @pl.loop(2, n_chunks, step=2)
    def _(t):
        for b in range(2):
            g = t + b
            wait_idx(b)
            wait_store(b)
            fire_gathers(g, b)
            wait_gathers(1 - b)
            store(g - 1, 1 - b)
            r0 = jnp.minimum(idx_start(g + 1), base + (rows_per_w - _K))
            pltpu.async_copy(idx_hbm.at[pl.ds(r0, _K)], idx_b[1 - b],
                             isem[1 - b])

    b_last = (n_chunks - 1) % 2
    wait_gathers(b_last)
    store(n_chunks - 1, b_last)
    wait_store(1 - b_last)
    wait_store(b_last)
    wait_idx(1 - b_last)


@jax.jit
def _embedding_lookup(idx2, weight):
    n_rows = idx2.shape[0]
    d = weight.shape[1]
    run = pl.kernel(
        _gather_body,
        out_type=jax.ShapeDtypeStruct((n_rows * _G, d), jnp.float32),
        mesh=plsc.VectorSubcoreMesh(
            core_axis_name="c", subcore_axis_name="s",
            num_cores=_NC, num_subcores=_NS),
        scratch_types=[
            pltpu.VMEM((_K, _G), jnp.int32),
            pltpu.VMEM((_K, _G), jnp.int32),
            pltpu.VMEM((_K * _G, d), jnp.float32),
            pltpu.VMEM((_K * _G, d), jnp.float32),
            pltpu.SemaphoreType.DMA,
            pltpu.SemaphoreType.DMA,
            pltpu.SemaphoreType.DMA,
            pltpu.SemaphoreType.DMA,
            pltpu.SemaphoreType.DMA,
            pltpu.SemaphoreType.DMA,
        ],
        compiler_params=pltpu.CompilerParams(use_tc_tiling_on_sc=False),
    )
    return run(idx2, weight)


def kernel(x, weight):
    b, s = x.shape
    d = weight.shape[1]
    idx2 = x.reshape(-1, _G).astype(jnp.int32)
    out = _embedding_lookup(idx2, weight)
    return out.reshape(b, s, d)
